# Initial kernel scaffold; baseline (speedup 1.0000x reference)
import jax
import jax.numpy as jnp
from jax.experimental import pallas as pl


def kernel(x, edge_index, W1, b1, W2, b2, Wl, bl):
    raise NotImplementedError("write your pallas kernel here")


# speedup vs baseline: 11.5995x; 11.5995x over previous
import functools

import jax
import jax.numpy as jnp
from jax import lax
from jax.experimental import pallas as pl
from jax.experimental.pallas import tpu as pltpu
from jax.experimental.pallas import tpu_sc as plsc

N = 100000
E = 1600000
IN_C, HID_C, OUT_C = 9, 32, 4

NC, NS = 2, 16
CH = 128
N_PAD = 102400
HALF = N_PAD // 2
GARB = NS * CH
ROWS = HALF + GARB
E_PAD = 1601536

_SC_PARAMS = pltpu.CompilerParams(use_tc_tiling_on_sc=False)


def _deg_body(dst_hbm, degp_hbm, acc, dstb, onesb, zb):
    c = lax.axis_index("c")
    s = lax.axis_index("s")

    def _zb(i, _):
        zb[pl.ds(i * 16, 16)] = jnp.zeros((16,), jnp.float32)
        return 0
    lax.fori_loop(0, zb.shape[0] // 16, _zb, 0)
    sl = N_PAD // NS
    pltpu.sync_copy(zb, acc.at[pl.ds(s * sl, sl)])
    for j in range(CH // 16):
        onesb[pl.ds(j * 16, 16)] = jnp.ones((16,), jnp.float32)
    plsc.subcore_barrier()

    per_w = E_PAD // (NC * NS)
    base = (c * NS + s) * per_w

    def _step(g, _):
        pltpu.sync_copy(dst_hbm.at[pl.ds(base + g * CH, CH)], dstb)
        pltpu.sync_copy(onesb, acc.at[dstb], add=True)
        return 0
    lax.fori_loop(0, per_w // CH, _step, 0)

    plsc.subcore_barrier()
    pltpu.sync_copy(acc.at[pl.ds(s * sl, sl)], degp_hbm.at[c, pl.ds(s * sl, sl)])


def _deg_partials(dst_pad):
    mesh = plsc.VectorSubcoreMesh(core_axis_name="c", subcore_axis_name="s")
    return pl.kernel(
        _deg_body,
        out_type=jax.ShapeDtypeStruct((NC, N_PAD), jnp.float32),
        mesh=mesh,
        scratch_types=[
            pltpu.VMEM_SHARED((N_PAD,), jnp.float32),
            pltpu.VMEM((CH,), jnp.int32),
            pltpu.VMEM((CH,), jnp.float32),
            pltpu.VMEM((N_PAD // NS,), jnp.float32),
        ],
        compiler_params=_SC_PARAMS,
    )(dst_pad)


def _scat_body(g_hbm, src_hbm, dst_hbm, out_hbm, acc, srcb, dstb, sidx, rows,
               zb, gsem, D):
    c = lax.axis_index("c")
    s = lax.axis_index("s")
    lo = c * HALF

    zr = zb.shape[0]

    def _zb(i, _):
        for k in range(D // 16):
            zb[i, pl.ds(k * 16, 16)] = jnp.zeros((16,), jnp.float32)
        return 0
    lax.fori_loop(0, zr, _zb, 0)
    tile_rows = ROWS // NS

    def _zc(i, _):
        pltpu.sync_copy(zb, acc.at[pl.ds(s * tile_rows + i * zr, zr)])
        return 0
    lax.fori_loop(0, tile_rows // zr, _zc, 0)
    plsc.subcore_barrier()

    per_t = E_PAD // NS
    base = s * per_t
    garb0 = HALF + s * CH
    iota16 = lax.iota(jnp.int32, 16)

    def _step(g, _):
        e0 = base + g * CH
        pltpu.sync_copy(src_hbm.at[pl.ds(e0, CH)], srcb)
        pltpu.sync_copy(dst_hbm.at[pl.ds(e0, CH)], dstb)
        pltpu.async_copy(g_hbm.at[srcb], rows, gsem).wait()
        for j in range(CH // 16):
            d = dstb[pl.ds(j * 16, 16)]
            local = d - lo
            ok = (local >= 0) & (local < HALF)
            garb = garb0 + ((j * 16 + iota16) % CH)
            sidx[pl.ds(j * 16, 16)] = jnp.where(ok, local, garb)
        pltpu.sync_copy(rows, acc.at[sidx], add=True)
        return 0
    lax.fori_loop(0, per_t // CH, _step, 0)

    plsc.subcore_barrier()
    out_rows = HALF // NS
    pltpu.sync_copy(acc.at[pl.ds(s * out_rows, out_rows)],
                    out_hbm.at[pl.ds(lo + s * out_rows, out_rows)])


def _scatter_rows(g, src_pad, dst_pad):
    D = g.shape[1]
    mesh = plsc.VectorSubcoreMesh(core_axis_name="c", subcore_axis_name="s")
    zr = 416
    return pl.kernel(
        functools.partial(_scat_body, D=D),
        out_type=jax.ShapeDtypeStruct((N_PAD, D), jnp.float32),
        mesh=mesh,
        scratch_types=[
            pltpu.VMEM_SHARED((ROWS, D), jnp.float32),
            pltpu.VMEM((CH,), jnp.int32),
            pltpu.VMEM((CH,), jnp.int32),
            pltpu.VMEM((CH,), jnp.int32),
            pltpu.VMEM((CH, D), jnp.float32),
            pltpu.VMEM((zr, D), jnp.float32),
            pltpu.SemaphoreType.DMA,
        ],
        compiler_params=_SC_PARAMS,
    )(g, src_pad, dst_pad)


_BLK = 5120
_GRID = N_PAD // _BLK


def _pre_body(degT_ref, x_ref, gx_ref, dis_ref):
    deg = degT_ref[:, 0:1] + degT_ref[:, 1:2] + 1.0
    dis = 1.0 / jnp.sqrt(deg)
    dis_ref[...] = dis
    gx_ref[...] = x_ref[...] * dis


def _pre(degT, x_pad):
    return pl.pallas_call(
        _pre_body,
        grid=(_GRID,),
        in_specs=[
            pl.BlockSpec((_BLK, 2), lambda i: (i, 0)),
            pl.BlockSpec((_BLK, 16), lambda i: (i, 0)),
        ],
        out_specs=[
            pl.BlockSpec((_BLK, 16), lambda i: (i, 0)),
            pl.BlockSpec((_BLK, 1), lambda i: (i, 0)),
        ],
        out_shape=[
            jax.ShapeDtypeStruct((N_PAD, 16), jnp.float32),
            jax.ShapeDtypeStruct((N_PAD, 1), jnp.float32),
        ],
    )(degT, x_pad)


def _l1post_body(S1_ref, gx_ref, dis_ref, W_ref, b_ref, g2_ref):
    agg = dis_ref[...] * (S1_ref[...] + gx_ref[...])
    h = jnp.maximum(
        jnp.dot(agg, W_ref[...], preferred_element_type=jnp.float32)
        + b_ref[...], 0.0)
    g2_ref[...] = dis_ref[...] * h


def _l1post(S1, gx, dis, W1p, b1r):
    return pl.pallas_call(
        _l1post_body,
        grid=(_GRID,),
        in_specs=[
            pl.BlockSpec((_BLK, 16), lambda i: (i, 0)),
            pl.BlockSpec((_BLK, 16), lambda i: (i, 0)),
            pl.BlockSpec((_BLK, 1), lambda i: (i, 0)),
            pl.BlockSpec((16, HID_C), lambda i: (0, 0)),
            pl.BlockSpec((1, HID_C), lambda i: (0, 0)),
        ],
        out_specs=pl.BlockSpec((_BLK, HID_C), lambda i: (i, 0)),
        out_shape=jax.ShapeDtypeStruct((N_PAD, HID_C), jnp.float32),
    )(S1, gx, dis, W1p, b1r)


def _l2post_body(S2_ref, g2_ref, dis_ref, W2_ref, b2_ref, Wl_ref, bl_ref,
                 out_ref):
    agg = dis_ref[...] * (S2_ref[...] + g2_ref[...])
    h = jnp.maximum(
        jnp.dot(agg, W2_ref[...], preferred_element_type=jnp.float32)
        + b2_ref[...], 0.0)
    out_ref[...] = (
        jnp.dot(h, Wl_ref[...], preferred_element_type=jnp.float32)
        + bl_ref[...])


def _l2post(S2, g2, dis, W2, b2r, Wl, blr):
    return pl.pallas_call(
        _l2post_body,
        grid=(_GRID,),
        in_specs=[
            pl.BlockSpec((_BLK, HID_C), lambda i: (i, 0)),
            pl.BlockSpec((_BLK, HID_C), lambda i: (i, 0)),
            pl.BlockSpec((_BLK, 1), lambda i: (i, 0)),
            pl.BlockSpec((HID_C, HID_C), lambda i: (0, 0)),
            pl.BlockSpec((1, HID_C), lambda i: (0, 0)),
            pl.BlockSpec((HID_C, OUT_C), lambda i: (0, 0)),
            pl.BlockSpec((1, OUT_C), lambda i: (0, 0)),
        ],
        out_specs=pl.BlockSpec((_BLK, OUT_C), lambda i: (i, 0)),
        out_shape=jax.ShapeDtypeStruct((N_PAD, OUT_C), jnp.float32),
    )(S2, g2, dis, W2, b2r, Wl, blr)


def kernel(x, edge_index, W1, b1, W2, b2, Wl, bl):
    pe = E_PAD - E
    src = jnp.concatenate(
        [edge_index[0], (jnp.arange(pe, dtype=jnp.int32) * 97) % N])
    dst = jnp.concatenate(
        [edge_index[1], N + (jnp.arange(pe, dtype=jnp.int32) % 64)])

    x_pad = jnp.pad(x, ((0, N_PAD - N), (0, 16 - IN_C)))
    W1p = jnp.pad(W1, ((0, 16 - IN_C), (0, 0)))

    degp = _deg_partials(dst)
    gx, dis = _pre(degp.T, x_pad)

    S1 = _scatter_rows(gx, src, dst)
    g2 = _l1post(S1, gx, dis, W1p, b1.reshape(1, HID_C))

    S2 = _scatter_rows(g2, src, dst)
    out = _l2post(S2, g2, dis, W2, b2.reshape(1, HID_C),
                  Wl, bl.reshape(1, OUT_C))
    return out[:N]


# speedup vs baseline: 34.3508x; 2.9614x over previous
import functools

import jax
import jax.numpy as jnp
from jax import lax
from jax.experimental import pallas as pl
from jax.experimental.pallas import tpu as pltpu
from jax.experimental.pallas import tpu_sc as plsc

N = 100000
E = 1600000
IN_C, HID_C, OUT_C = 9, 32, 4

NC, NS = 2, 16
CH = 128
N_PAD = 100352
HALF = N_PAD // 2
GARB = NS * CH
ROWS = HALF + GARB
E_PAD = 1605632
NB = 8

_SC_PARAMS = pltpu.CompilerParams(use_tc_tiling_on_sc=False)


def _deg_body(dst_hbm, degp_hbm, acc, dstb, onesb, zb, isem, asem):
    c = lax.axis_index("c")
    s = lax.axis_index("s")

    def _zb(i, _):
        zb[pl.ds(i * 16, 16)] = jnp.zeros((16,), jnp.float32)
        return 0
    lax.fori_loop(0, zb.shape[0] // 16, _zb, 0)
    sl = N_PAD // NS
    pltpu.sync_copy(zb, acc.at[pl.ds(s * sl, sl)])
    for j in range(CH // 16):
        onesb[pl.ds(j * 16, 16)] = jnp.ones((16,), jnp.float32)
    plsc.subcore_barrier()

    per_w = E_PAD // (NC * NS)
    base = (c * NS + s) * per_w

    def _burst(o, _):
        e0 = base + o * (NB * CH)
        din = [pltpu.async_copy(dst_hbm.at[pl.ds(e0 + b * CH, CH)],
                                dstb.at[b], isem.at[b]) for b in range(NB)]
        dadd = []
        for b in range(NB):
            din[b].wait()
            dadd.append(pltpu.async_copy(onesb, acc.at[dstb.at[b]],
                                         asem.at[b], add=True))
        for b in range(NB):
            dadd[b].wait()
        return 0
    lax.fori_loop(0, per_w // (NB * CH), _burst, 0)

    plsc.subcore_barrier()
    pltpu.sync_copy(acc.at[pl.ds(s * sl, sl)], degp_hbm.at[c, pl.ds(s * sl, sl)])


def _deg_partials(dst_pad):
    mesh = plsc.VectorSubcoreMesh(core_axis_name="c", subcore_axis_name="s")
    return pl.kernel(
        _deg_body,
        out_type=jax.ShapeDtypeStruct((NC, N_PAD), jnp.float32),
        mesh=mesh,
        scratch_types=[
            pltpu.VMEM_SHARED((N_PAD,), jnp.float32),
            pltpu.VMEM((NB, CH), jnp.int32),
            pltpu.VMEM((CH,), jnp.float32),
            pltpu.VMEM((N_PAD // NS,), jnp.float32),
            pltpu.SemaphoreType.DMA((NB,)),
            pltpu.SemaphoreType.DMA((NB,)),
        ],
        compiler_params=_SC_PARAMS,
    )(dst_pad)


def _scat_body(g_hbm, src_hbm, dst_hbm, out_hbm, acc, srcb, dstb, sidx, rows,
               zb, isem, gsem, ssem, D, nb):
    c = lax.axis_index("c")
    s = lax.axis_index("s")
    lo = c * HALF

    zr = zb.shape[0]

    def _zb(i, _):
        for k in range(D // 16):
            zb[i, pl.ds(k * 16, 16)] = jnp.zeros((16,), jnp.float32)
        return 0
    lax.fori_loop(0, zr, _zb, 0)
    tile_rows = ROWS // NS

    def _zc(i, _):
        pltpu.sync_copy(zb, acc.at[pl.ds(s * tile_rows + i * zr, zr)])
        return 0
    lax.fori_loop(0, tile_rows // zr, _zc, 0)
    plsc.subcore_barrier()

    per_t = E_PAD // NS
    base = s * per_t
    garb0 = HALF + s * CH
    iota16 = lax.iota(jnp.int32, 16)

    def _burst(o, _):
        e0 = base + o * (nb * CH)
        din = []
        for b in range(nb):
            eb = e0 + b * CH
            din.append((
                pltpu.async_copy(src_hbm.at[pl.ds(eb, CH)], srcb.at[b],
                                 isem.at[b]),
                pltpu.async_copy(dst_hbm.at[pl.ds(eb, CH)], dstb.at[b],
                                 isem.at[b]),
            ))
        dg = []
        for b in range(nb):
            din[b][0].wait()
            din[b][1].wait()
            dg.append(pltpu.async_copy(g_hbm.at[srcb.at[b]], rows.at[b],
                                       gsem.at[b]))
        for b in range(nb):
            for j in range(CH // 16):
                d = dstb[b, pl.ds(j * 16, 16)]
                local = d - lo
                ok = (local >= 0) & (local < HALF)
                garb = garb0 + ((j * 16 + iota16) % CH)
                sidx[b, pl.ds(j * 16, 16)] = jnp.where(ok, local, garb)
        ds_ = []
        for b in range(nb):
            dg[b].wait()
            ds_.append(pltpu.async_copy(rows.at[b], acc.at[sidx.at[b]],
                                        ssem.at[b], add=True))
        for b in range(nb):
            ds_[b].wait()
        return 0
    lax.fori_loop(0, per_t // (nb * CH), _burst, 0)

    plsc.subcore_barrier()
    out_rows = HALF // NS
    pltpu.sync_copy(acc.at[pl.ds(s * out_rows, out_rows)],
                    out_hbm.at[pl.ds(lo + s * out_rows, out_rows)])


def _scatter_rows(g, src_pad, dst_pad):
    D = g.shape[1]
    nb = 4 if D == 32 else NB
    mesh = plsc.VectorSubcoreMesh(core_axis_name="c", subcore_axis_name="s")
    zr = 102
    return pl.kernel(
        functools.partial(_scat_body, D=D, nb=nb),
        out_type=jax.ShapeDtypeStruct((N_PAD, D), jnp.float32),
        mesh=mesh,
        scratch_types=[
            pltpu.VMEM_SHARED((ROWS, D), jnp.float32),
            pltpu.VMEM((nb, CH), jnp.int32),
            pltpu.VMEM((nb, CH), jnp.int32),
            pltpu.VMEM((nb, CH), jnp.int32),
            pltpu.VMEM((nb, CH, D), jnp.float32),
            pltpu.VMEM((zr, D), jnp.float32),
            pltpu.SemaphoreType.DMA((nb,)),
            pltpu.SemaphoreType.DMA((nb,)),
            pltpu.SemaphoreType.DMA((nb,)),
        ],
        compiler_params=_SC_PARAMS,
    )(g, src_pad, dst_pad)


_BLK = 6272
_GRID = N_PAD // _BLK


def _pre_body(degT_ref, x_ref, gx_ref, dis_ref):
    deg = degT_ref[:, 0:1] + degT_ref[:, 1:2] + 1.0
    dis = 1.0 / jnp.sqrt(deg)
    dis_ref[...] = dis
    gx_ref[...] = x_ref[...] * dis


def _pre(degT, x_pad):
    return pl.pallas_call(
        _pre_body,
        grid=(_GRID,),
        in_specs=[
            pl.BlockSpec((_BLK, 2), lambda i: (i, 0)),
            pl.BlockSpec((_BLK, 16), lambda i: (i, 0)),
        ],
        out_specs=[
            pl.BlockSpec((_BLK, 16), lambda i: (i, 0)),
            pl.BlockSpec((_BLK, 1), lambda i: (i, 0)),
        ],
        out_shape=[
            jax.ShapeDtypeStruct((N_PAD, 16), jnp.float32),
            jax.ShapeDtypeStruct((N_PAD, 1), jnp.float32),
        ],
    )(degT, x_pad)


def _l1post_body(S1_ref, gx_ref, dis_ref, W_ref, b_ref, g2_ref):
    agg = dis_ref[...] * (S1_ref[...] + gx_ref[...])
    h = jnp.maximum(
        jnp.dot(agg, W_ref[...], preferred_element_type=jnp.float32)
        + b_ref[...], 0.0)
    g2_ref[...] = dis_ref[...] * h


def _l1post(S1, gx, dis, W1p, b1r):
    return pl.pallas_call(
        _l1post_body,
        grid=(_GRID,),
        in_specs=[
            pl.BlockSpec((_BLK, 16), lambda i: (i, 0)),
            pl.BlockSpec((_BLK, 16), lambda i: (i, 0)),
            pl.BlockSpec((_BLK, 1), lambda i: (i, 0)),
            pl.BlockSpec((16, HID_C), lambda i: (0, 0)),
            pl.BlockSpec((1, HID_C), lambda i: (0, 0)),
        ],
        out_specs=pl.BlockSpec((_BLK, HID_C), lambda i: (i, 0)),
        out_shape=jax.ShapeDtypeStruct((N_PAD, HID_C), jnp.float32),
    )(S1, gx, dis, W1p, b1r)


def _l2post_body(S2_ref, g2_ref, dis_ref, W2_ref, b2_ref, Wl_ref, bl_ref,
                 out_ref):
    agg = dis_ref[...] * (S2_ref[...] + g2_ref[...])
    h = jnp.maximum(
        jnp.dot(agg, W2_ref[...], preferred_element_type=jnp.float32)
        + b2_ref[...], 0.0)
    out_ref[...] = (
        jnp.dot(h, Wl_ref[...], preferred_element_type=jnp.float32)
        + bl_ref[...])


def _l2post(S2, g2, dis, W2, b2r, Wl, blr):
    return pl.pallas_call(
        _l2post_body,
        grid=(_GRID,),
        in_specs=[
            pl.BlockSpec((_BLK, HID_C), lambda i: (i, 0)),
            pl.BlockSpec((_BLK, HID_C), lambda i: (i, 0)),
            pl.BlockSpec((_BLK, 1), lambda i: (i, 0)),
            pl.BlockSpec((HID_C, HID_C), lambda i: (0, 0)),
            pl.BlockSpec((1, HID_C), lambda i: (0, 0)),
            pl.BlockSpec((HID_C, OUT_C), lambda i: (0, 0)),
            pl.BlockSpec((1, OUT_C), lambda i: (0, 0)),
        ],
        out_specs=pl.BlockSpec((_BLK, OUT_C), lambda i: (i, 0)),
        out_shape=jax.ShapeDtypeStruct((N_PAD, OUT_C), jnp.float32),
    )(S2, g2, dis, W2, b2r, Wl, blr)


def kernel(x, edge_index, W1, b1, W2, b2, Wl, bl):
    pe = E_PAD - E
    src = jnp.concatenate(
        [edge_index[0], (jnp.arange(pe, dtype=jnp.int32) * 97) % N])
    dst = jnp.concatenate(
        [edge_index[1], N + (jnp.arange(pe, dtype=jnp.int32) % 64)])

    x_pad = jnp.pad(x, ((0, N_PAD - N), (0, 16 - IN_C)))
    W1p = jnp.pad(W1, ((0, 16 - IN_C), (0, 0)))

    degp = _deg_partials(dst)
    gx, dis = _pre(degp.T, x_pad)

    S1 = _scatter_rows(gx, src, dst)
    g2 = _l1post(S1, gx, dis, W1p, b1.reshape(1, HID_C))

    S2 = _scatter_rows(g2, src, dst)
    out = _l2post(S2, g2, dis, W2, b2.reshape(1, HID_C),
                  Wl, bl.reshape(1, OUT_C))
    return out[:N]


# speedup vs baseline: 40.6175x; 1.1824x over previous
import functools

import jax
import jax.numpy as jnp
from jax import lax
from jax.experimental import pallas as pl
from jax.experimental.pallas import tpu as pltpu
from jax.experimental.pallas import tpu_sc as plsc

N = 100000
E = 1600000
IN_C, HID_C, OUT_C = 9, 32, 4

NC, NS = 2, 16
CH = 128
N_PAD = 100352
HALF = N_PAD // 2
GARB = NS * CH
ROWS = HALF + GARB
NB = 8

ECH = E // CH
TCH, TXT = ECH // NS, ECH % NS
WCH, WXT = ECH // (NC * NS), ECH % (NC * NS)

X16 = N_PAD * 16 // 128
X32 = N_PAD * 32 // 128

_SC_PARAMS = pltpu.CompilerParams(use_tc_tiling_on_sc=False)


def _deg_body(dst_hbm, degp_hbm, acc, dstb, onesb, zb, isem, asem):
    c = lax.axis_index("c")
    s = lax.axis_index("s")
    w = c * NS + s

    def _zb(i, _):
        zb[pl.ds(i * 16, 16)] = jnp.zeros((16,), jnp.float32)
        return 0
    lax.fori_loop(0, zb.shape[0] // 16, _zb, 0)
    sl = N_PAD // NS
    pltpu.sync_copy(zb, acc.at[pl.ds(s * sl, sl)])
    for j in range(CH // 16):
        onesb[pl.ds(j * 16, 16)] = jnp.ones((16,), jnp.float32)
    plsc.subcore_barrier()

    base = w * WCH * CH

    def _burst(e0, m):
        din = [pltpu.async_copy(dst_hbm.at[pl.ds(e0 + b * CH, CH)],
                                dstb.at[b], isem.at[b]) for b in range(m)]
        dadd = []
        for b in range(m):
            din[b].wait()
            dadd.append(pltpu.async_copy(onesb, acc.at[dstb.at[b]],
                                         asem.at[b], add=True))
        for b in range(m):
            dadd[b].wait()

    r = WCH % NB
    if r:
        _burst(base, r)

    def _full(o, _):
        _burst(base + (r + o * NB) * CH, NB)
        return 0
    lax.fori_loop(0, WCH // NB, _full, 0)

    @pl.when(w < WXT)
    def _extra():
        _burst(((NC * NS) * WCH + w) * CH, 1)

    plsc.subcore_barrier()
    pltpu.sync_copy(acc.at[pl.ds(s * sl, sl)], degp_hbm.at[c, pl.ds(s * sl, sl)])


def _deg_partials(dst):
    mesh = plsc.VectorSubcoreMesh(core_axis_name="c", subcore_axis_name="s")
    return pl.kernel(
        _deg_body,
        out_type=jax.ShapeDtypeStruct((NC, N_PAD), jnp.float32),
        mesh=mesh,
        scratch_types=[
            pltpu.VMEM_SHARED((N_PAD,), jnp.float32),
            pltpu.VMEM((NB, CH), jnp.int32),
            pltpu.VMEM((CH,), jnp.float32),
            pltpu.VMEM((N_PAD // NS,), jnp.float32),
            pltpu.SemaphoreType.DMA((NB,)),
            pltpu.SemaphoreType.DMA((NB,)),
        ],
        compiler_params=_SC_PARAMS,
    )(dst)


def _scat_body(g_hbm, src_hbm, dst_hbm, out_hbm, acc, srcb, dstb, sidx, rows,
               zb, isem, gsem, ssem, D, nb):
    c = lax.axis_index("c")
    s = lax.axis_index("s")
    lo = c * HALF

    zr = zb.shape[0]

    def _zb(i, _):
        for k in range(D // 16):
            zb[i, pl.ds(k * 16, 16)] = jnp.zeros((16,), jnp.float32)
        return 0
    lax.fori_loop(0, zr, _zb, 0)
    tile_rows = ROWS // NS

    def _zc(i, _):
        pltpu.sync_copy(zb, acc.at[pl.ds(s * tile_rows + i * zr, zr)])
        return 0
    lax.fori_loop(0, tile_rows // zr, _zc, 0)
    plsc.subcore_barrier()

    garb0 = HALF + s * CH
    iota16 = lax.iota(jnp.int32, 16)
    base = s * TCH * CH

    def _burst(e0, m):
        din = []
        for b in range(m):
            eb = e0 + b * CH
            din.append((
                pltpu.async_copy(src_hbm.at[pl.ds(eb, CH)], srcb.at[b],
                                 isem.at[b]),
                pltpu.async_copy(dst_hbm.at[pl.ds(eb, CH)], dstb.at[b],
                                 isem.at[b]),
            ))
        dg = []
        for b in range(m):
            din[b][0].wait()
            din[b][1].wait()
            dg.append(pltpu.async_copy(g_hbm.at[srcb.at[b]], rows.at[b],
                                       gsem.at[b]))
        for b in range(m):
            for j in range(CH // 16):
                d = dstb[b, pl.ds(j * 16, 16)]
                local = d - lo
                ok = (local >= 0) & (local < HALF)
                garb = garb0 + ((j * 16 + iota16) % CH)
                sidx[b, pl.ds(j * 16, 16)] = jnp.where(ok, local, garb)
        ds_ = []
        for b in range(m):
            dg[b].wait()
            ds_.append(pltpu.async_copy(rows.at[b], acc.at[sidx.at[b]],
                                        ssem.at[b], add=True))
        for b in range(m):
            ds_[b].wait()

    r = TCH % nb
    if r:
        _burst(base, r)

    def _full(o, _):
        _burst(base + (r + o * nb) * CH, nb)
        return 0
    lax.fori_loop(0, TCH // nb, _full, 0)

    @pl.when(s < TXT)
    def _extra():
        _burst((NS * TCH + s) * CH, 1)

    plsc.subcore_barrier()
    out_rows = HALF // NS
    pltpu.sync_copy(acc.at[pl.ds(s * out_rows, out_rows)],
                    out_hbm.at[pl.ds(lo + s * out_rows, out_rows)])


def _scatter_rows(g, src, dst):
    D = g.shape[1]
    nb = 4 if D == 32 else NB
    mesh = plsc.VectorSubcoreMesh(core_axis_name="c", subcore_axis_name="s")
    zr = 102
    return pl.kernel(
        functools.partial(_scat_body, D=D, nb=nb),
        out_type=jax.ShapeDtypeStruct((N_PAD, D), jnp.float32),
        mesh=mesh,
        scratch_types=[
            pltpu.VMEM_SHARED((ROWS, D), jnp.float32),
            pltpu.VMEM((nb, CH), jnp.int32),
            pltpu.VMEM((nb, CH), jnp.int32),
            pltpu.VMEM((nb, CH), jnp.int32),
            pltpu.VMEM((nb, CH, D), jnp.float32),
            pltpu.VMEM((zr, D), jnp.float32),
            pltpu.SemaphoreType.DMA((nb,)),
            pltpu.SemaphoreType.DMA((nb,)),
            pltpu.SemaphoreType.DMA((nb,)),
        ],
        compiler_params=_SC_PARAMS,
    )(g, src, dst)


_GRID = 16
_NBLK = N_PAD // _GRID
_B16 = X16 // _GRID
_B32 = X32 // _GRID


def _dis_body(degp_ref, dis_ref):
    deg = degp_ref[0:1, :] + degp_ref[1:2, :] + 1.0
    dis_ref[...] = 1.0 / jnp.sqrt(deg)


def _dis(degp):
    return pl.pallas_call(
        _dis_body,
        grid=(_GRID,),
        in_specs=[pl.BlockSpec((2, _NBLK), lambda i: (0, i))],
        out_specs=pl.BlockSpec((1, _NBLK), lambda i: (0, i)),
        out_shape=jax.ShapeDtypeStruct((1, N_PAD), jnp.float32),
    )(degp)


def _gx_body(x_ref, rep_ref, gx_ref):
    gx_ref[...] = x_ref[...] * rep_ref[...]


def _gx(x_flat, rep16):
    return pl.pallas_call(
        _gx_body,
        grid=(_GRID,),
        in_specs=[
            pl.BlockSpec((_B16, 128), lambda i: (i, 0)),
            pl.BlockSpec((_B16, 128), lambda i: (i, 0)),
        ],
        out_specs=pl.BlockSpec((_B16, 128), lambda i: (i, 0)),
        out_shape=jax.ShapeDtypeStruct((X16, 128), jnp.float32),
    )(x_flat, rep16)


def _l1post_body(S1_ref, gx_ref, r16_ref, r32_ref, W_ref, b_ref, g2_ref):
    t = r16_ref[...] * (S1_ref[...] + gx_ref[...])
    h = jnp.maximum(
        jnp.dot(t, W_ref[...], preferred_element_type=jnp.float32)
        + b_ref[...], 0.0)
    g2_ref[...] = r32_ref[...] * h


def _l1post(S1f, gxf, rep16, rep32w, Wb1, bb1):
    return pl.pallas_call(
        _l1post_body,
        grid=(_GRID,),
        in_specs=[
            pl.BlockSpec((_B16, 128), lambda i: (i, 0)),
            pl.BlockSpec((_B16, 128), lambda i: (i, 0)),
            pl.BlockSpec((_B16, 128), lambda i: (i, 0)),
            pl.BlockSpec((_B16, 256), lambda i: (i, 0)),
            pl.BlockSpec((128, 256), lambda i: (0, 0)),
            pl.BlockSpec((1, 256), lambda i: (0, 0)),
        ],
        out_specs=pl.BlockSpec((_B16, 256), lambda i: (i, 0)),
        out_shape=jax.ShapeDtypeStruct((X16, 256), jnp.float32),
    )(S1f, gxf, rep16, rep32w, Wb1, bb1)


def _l2post_body(S2_ref, g2_ref, r32_ref, W2_ref, b2_ref, Wl_ref, bl_ref,
                 out_ref):
    u = r32_ref[...] * (S2_ref[...] + g2_ref[...])
    h = jnp.maximum(
        jnp.dot(u, W2_ref[...], preferred_element_type=jnp.float32)
        + b2_ref[...], 0.0)
    out_ref[...] = (
        jnp.dot(h, Wl_ref[...], preferred_element_type=jnp.float32)
        + bl_ref[...])


def _l2post(S2f, g2f, rep32, Wb2, bb2, Wb3, bb3):
    return pl.pallas_call(
        _l2post_body,
        grid=(_GRID,),
        in_specs=[
            pl.BlockSpec((_B32, 128), lambda i: (i, 0)),
            pl.BlockSpec((_B32, 128), lambda i: (i, 0)),
            pl.BlockSpec((_B32, 128), lambda i: (i, 0)),
            pl.BlockSpec((128, 128), lambda i: (0, 0)),
            pl.BlockSpec((1, 128), lambda i: (0, 0)),
            pl.BlockSpec((128, 16), lambda i: (0, 0)),
            pl.BlockSpec((1, 16), lambda i: (0, 0)),
        ],
        out_specs=pl.BlockSpec((_B32, 16), lambda i: (i, 0)),
        out_shape=jax.ShapeDtypeStruct((X32, 16), jnp.float32),
    )(S2f, g2f, rep32, Wb2, bb2, Wb3, bb3)


def _block_diag(W, k):
    m, n = W.shape
    out = jnp.zeros((k * m, k * n), W.dtype)
    for i in range(k):
        out = out.at[i * m:(i + 1) * m, i * n:(i + 1) * n].set(W)
    return out


def kernel(x, edge_index, W1, b1, W2, b2, Wl, bl):
    src, dst = edge_index[0], edge_index[1]

    x_flat = jnp.pad(x, ((0, N_PAD - N), (0, 16 - IN_C))).reshape(X16, 128)
    W1p = jnp.pad(W1, ((0, 16 - IN_C), (0, 0)))
    Wb1 = _block_diag(W1p, 8)
    bb1 = jnp.tile(b1, 8).reshape(1, 256)
    Wb2 = _block_diag(W2, 4)
    bb2 = jnp.tile(b2, 4).reshape(1, 128)
    Wb3 = _block_diag(Wl, 4)
    bb3 = jnp.tile(bl, 4).reshape(1, 16)

    degp = _deg_partials(dst)
    dis = _dis(degp).reshape(N_PAD)
    rep16 = jnp.broadcast_to(dis[:, None], (N_PAD, 16)).reshape(X16, 128)
    rep32 = jnp.broadcast_to(dis[:, None], (N_PAD, 32)).reshape(X32, 128)

    gxf = _gx(x_flat, rep16)
    S1 = _scatter_rows(gxf.reshape(N_PAD, 16), src, dst)
    g2 = _l1post(S1.reshape(X16, 128), gxf, rep16,
                 rep32.reshape(X16, 256), Wb1, bb1)

    S2 = _scatter_rows(g2.reshape(N_PAD, 32), src, dst)
    out = _l2post(S2.reshape(X32, 128), g2.reshape(X32, 128), rep32,
                  Wb2, bb2, Wb3, bb3)
    return out.reshape(N_PAD, 4)[:N]


# speedup vs baseline: 47.5085x; 1.1697x over previous
import functools

import jax
import jax.numpy as jnp
from jax import lax
from jax.experimental import pallas as pl
from jax.experimental.pallas import tpu as pltpu
from jax.experimental.pallas import tpu_sc as plsc

N = 100000
E = 1600000
IN_C, HID_C, OUT_C = 9, 32, 4

NC, NS = 2, 16
CH = 128
N_PAD = 100352
NB = 8

ECH = E // CH
WCH, WXT = ECH // (NC * NS), ECH % (NC * NS)
CPS = ECH // NC
TCH, TXT = CPS // NS, CPS % NS

X16 = N_PAD * 16 // 128
X32 = N_PAD * 32 // 128

_SC_PARAMS = pltpu.CompilerParams(use_tc_tiling_on_sc=False)


def _src_off(chunk):
    return (2 * chunk) * CH


def _dst_off(chunk):
    return (2 * chunk + 1) * CH


def _deg_body(eil_hbm, degp_hbm, acc, dstb, onesb, zb, isem, asem):
    c = lax.axis_index("c")
    s = lax.axis_index("s")
    w = c * NS + s

    def _zb(i, _):
        zb[pl.ds(i * 16, 16)] = jnp.zeros((16,), jnp.float32)
        return 0
    lax.fori_loop(0, zb.shape[0] // 16, _zb, 0)
    sl = N_PAD // NS
    pltpu.sync_copy(zb, acc.at[pl.ds(s * sl, sl)])
    for j in range(CH // 16):
        onesb[pl.ds(j * 16, 16)] = jnp.ones((16,), jnp.float32)
    plsc.subcore_barrier()

    base = w * WCH

    def _burst(ch0, m):
        din = [pltpu.async_copy(eil_hbm.at[pl.ds(_dst_off(ch0 + b), CH)],
                                dstb.at[b], isem.at[b]) for b in range(m)]
        dadd = []
        for b in range(m):
            din[b].wait()
            dadd.append(pltpu.async_copy(onesb, acc.at[dstb.at[b]],
                                         asem.at[b], add=True))
        for b in range(m):
            dadd[b].wait()

    r = WCH % NB
    if r:
        _burst(base, r)

    def _full(o, _):
        _burst(base + r + o * NB, NB)
        return 0
    lax.fori_loop(0, WCH // NB, _full, 0)

    @pl.when(w < WXT)
    def _extra():
        _burst((NC * NS) * WCH + w, 1)

    plsc.subcore_barrier()
    pltpu.sync_copy(acc.at[pl.ds(s * sl, sl)], degp_hbm.at[c, pl.ds(s * sl, sl)])


def _deg_partials(eil):
    mesh = plsc.VectorSubcoreMesh(core_axis_name="c", subcore_axis_name="s")
    return pl.kernel(
        _deg_body,
        out_type=jax.ShapeDtypeStruct((NC, N_PAD), jnp.float32),
        mesh=mesh,
        scratch_types=[
            pltpu.VMEM_SHARED((N_PAD,), jnp.float32),
            pltpu.VMEM((NB, CH), jnp.int32),
            pltpu.VMEM((CH,), jnp.float32),
            pltpu.VMEM((N_PAD // NS,), jnp.float32),
            pltpu.SemaphoreType.DMA((NB,)),
            pltpu.SemaphoreType.DMA((NB,)),
        ],
        compiler_params=_SC_PARAMS,
    )(eil)


def _scat_body(g_hbm, eil_hbm, out_hbm, acc, srcb, dstb, rows, zb,
               isem, gsem, ssem, D, dt):
    c = lax.axis_index("c")
    s = lax.axis_index("s")

    zr = zb.shape[0]
    zvec = 32 if dt == jnp.bfloat16 else 16

    def _zb(i, _):
        for k in range(D // zvec):
            zb[i, pl.ds(k * zvec, zvec)] = jnp.zeros((zvec,), dt)
        return 0
    lax.fori_loop(0, zr, _zb, 0)
    sl = N_PAD // NS

    def _zc(i, _):
        pltpu.sync_copy(zb, acc.at[pl.ds(s * sl + i * zr, zr)])
        return 0
    lax.fori_loop(0, sl // zr, _zc, 0)
    plsc.subcore_barrier()

    base = c * CPS + s * TCH

    def _burst(ch0, m):
        din = []
        for b in range(m):
            din.append((
                pltpu.async_copy(eil_hbm.at[pl.ds(_src_off(ch0 + b), CH)],
                                 srcb.at[b], isem.at[b]),
                pltpu.async_copy(eil_hbm.at[pl.ds(_dst_off(ch0 + b), CH)],
                                 dstb.at[b], isem.at[b]),
            ))
        dg = []
        for b in range(m):
            din[b][0].wait()
            dg.append(pltpu.async_copy(g_hbm.at[srcb.at[b]], rows.at[b],
                                       gsem.at[b]))
        ds_ = []
        for b in range(m):
            din[b][1].wait()
            dg[b].wait()
            ds_.append(pltpu.async_copy(rows.at[b], acc.at[dstb.at[b]],
                                        ssem.at[b], add=True))
        for b in range(m):
            ds_[b].wait()

    r = TCH % NB
    if r:
        _burst(base, r)

    def _full(o, _):
        _burst(base + r + o * NB, NB)
        return 0
    lax.fori_loop(0, TCH // NB, _full, 0)

    @pl.when(s < TXT)
    def _extra():
        _burst(c * CPS + NS * TCH + s, 1)

    plsc.subcore_barrier()
    pltpu.sync_copy(acc.at[pl.ds(s * sl, sl)],
                    out_hbm.at[c, pl.ds(s * sl, sl)])


def _scatter_rows(g, eil):
    D = g.shape[1]
    dt = g.dtype
    mesh = plsc.VectorSubcoreMesh(core_axis_name="c", subcore_axis_name="s")
    zr = 98
    return pl.kernel(
        functools.partial(_scat_body, D=D, dt=dt),
        out_type=jax.ShapeDtypeStruct((NC, N_PAD, D), dt),
        mesh=mesh,
        scratch_types=[
            pltpu.VMEM_SHARED((N_PAD, D), dt),
            pltpu.VMEM((NB, CH), jnp.int32),
            pltpu.VMEM((NB, CH), jnp.int32),
            pltpu.VMEM((NB, CH, D), dt),
            pltpu.VMEM((zr, D), dt),
            pltpu.SemaphoreType.DMA((NB,)),
            pltpu.SemaphoreType.DMA((NB,)),
            pltpu.SemaphoreType.DMA((NB,)),
        ],
        compiler_params=_SC_PARAMS,
    )(g, eil)


_GRID = 16
_NBLK = N_PAD // _GRID
_B16 = X16 // _GRID
_B32 = X32 // _GRID


def _dis_body(degp_ref, dis_ref):
    deg = degp_ref[0:1, :] + degp_ref[1:2, :] + 1.0
    dis_ref[...] = 1.0 / jnp.sqrt(deg)


def _dis(degp):
    return pl.pallas_call(
        _dis_body,
        grid=(_GRID,),
        in_specs=[pl.BlockSpec((2, _NBLK), lambda i: (0, i))],
        out_specs=pl.BlockSpec((1, _NBLK), lambda i: (0, i)),
        out_shape=jax.ShapeDtypeStruct((1, N_PAD), jnp.float32),
    )(degp)


def _gx_body(x_ref, rep_ref, gx_ref):
    gx_ref[...] = x_ref[...] * rep_ref[...]


def _gx(x_flat, rep16):
    return pl.pallas_call(
        _gx_body,
        grid=(_GRID,),
        in_specs=[
            pl.BlockSpec((_B16, 128), lambda i: (i, 0)),
            pl.BlockSpec((_B16, 128), lambda i: (i, 0)),
        ],
        out_specs=pl.BlockSpec((_B16, 128), lambda i: (i, 0)),
        out_shape=jax.ShapeDtypeStruct((X16, 128), jnp.float32),
    )(x_flat, rep16)


def _l1post_body(S1a_ref, S1b_ref, gx_ref, r16_ref, r32_ref, W_ref, b_ref,
                 g2_ref):
    t = r16_ref[...] * (S1a_ref[...] + S1b_ref[...] + gx_ref[...])
    h = jnp.maximum(
        jnp.dot(t, W_ref[...], preferred_element_type=jnp.float32)
        + b_ref[...], 0.0)
    g2_ref[...] = (r32_ref[...] * h).astype(jnp.bfloat16)


def _l1post(S1a, S1b, gxf, rep16, rep32w, Wb1, bb1):
    return pl.pallas_call(
        _l1post_body,
        grid=(_GRID,),
        in_specs=[
            pl.BlockSpec((_B16, 128), lambda i: (i, 0)),
            pl.BlockSpec((_B16, 128), lambda i: (i, 0)),
            pl.BlockSpec((_B16, 128), lambda i: (i, 0)),
            pl.BlockSpec((_B16, 128), lambda i: (i, 0)),
            pl.BlockSpec((_B16, 256), lambda i: (i, 0)),
            pl.BlockSpec((128, 256), lambda i: (0, 0)),
            pl.BlockSpec((1, 256), lambda i: (0, 0)),
        ],
        out_specs=pl.BlockSpec((_B16, 256), lambda i: (i, 0)),
        out_shape=jax.ShapeDtypeStruct((X16, 256), jnp.bfloat16),
    )(S1a, S1b, gxf, rep16, rep32w, Wb1, bb1)


def _l2post_body(S2a_ref, S2b_ref, g2_ref, r32_ref, W2_ref, b2_ref, Wl_ref,
                 bl_ref, out_ref):
    agg = (S2a_ref[...].astype(jnp.float32) + S2b_ref[...].astype(jnp.float32)
           + g2_ref[...].astype(jnp.float32))
    u = r32_ref[...] * agg
    h = jnp.maximum(
        jnp.dot(u, W2_ref[...], preferred_element_type=jnp.float32)
        + b2_ref[...], 0.0)
    out_ref[...] = (
        jnp.dot(h, Wl_ref[...], preferred_element_type=jnp.float32)
        + bl_ref[...])


def _l2post(S2a, S2b, g2f, rep32, Wb2, bb2, Wb3, bb3):
    return pl.pallas_call(
        _l2post_body,
        grid=(_GRID,),
        in_specs=[
            pl.BlockSpec((_B32, 128), lambda i: (i, 0)),
            pl.BlockSpec((_B32, 128), lambda i: (i, 0)),
            pl.BlockSpec((_B32, 128), lambda i: (i, 0)),
            pl.BlockSpec((_B32, 128), lambda i: (i, 0)),
            pl.BlockSpec((128, 128), lambda i: (0, 0)),
            pl.BlockSpec((1, 128), lambda i: (0, 0)),
            pl.BlockSpec((128, 16), lambda i: (0, 0)),
            pl.BlockSpec((1, 16), lambda i: (0, 0)),
        ],
        out_specs=pl.BlockSpec((_B32, 16), lambda i: (i, 0)),
        out_shape=jax.ShapeDtypeStruct((X32, 16), jnp.float32),
    )(S2a, S2b, g2f, rep32, Wb2, bb2, Wb3, bb3)


def _block_diag(W, k):
    m, n = W.shape
    out = jnp.zeros((k * m, k * n), W.dtype)
    for i in range(k):
        out = out.at[i * m:(i + 1) * m, i * n:(i + 1) * n].set(W)
    return out


def kernel(x, edge_index, W1, b1, W2, b2, Wl, bl):
    eil = jnp.swapaxes(edge_index.reshape(2, ECH, CH), 0, 1).reshape(2 * E)

    x_flat = jnp.pad(x, ((0, N_PAD - N), (0, 16 - IN_C))).reshape(X16, 128)
    W1p = jnp.pad(W1, ((0, 16 - IN_C), (0, 0)))
    Wb1 = _block_diag(W1p, 8)
    bb1 = jnp.tile(b1, 8).reshape(1, 256)
    Wb2 = _block_diag(W2, 4)
    bb2 = jnp.tile(b2, 4).reshape(1, 128)
    Wb3 = _block_diag(Wl, 4)
    bb3 = jnp.tile(bl, 4).reshape(1, 16)

    degp = _deg_partials(eil)
    dis = _dis(degp).reshape(N_PAD)
    rep16 = jnp.broadcast_to(dis[:, None], (N_PAD, 16)).reshape(X16, 128)
    rep32 = jnp.broadcast_to(dis[:, None], (N_PAD, 32)).reshape(X32, 128)

    gxf = _gx(x_flat, rep16)
    S1 = _scatter_rows(gxf.reshape(N_PAD, 16), eil)
    S1f = S1.reshape(NC, X16, 128)
    g2 = _l1post(S1f[0], S1f[1], gxf, rep16,
                 rep32.reshape(X16, 256), Wb1, bb1)

    S2 = _scatter_rows(g2.reshape(N_PAD, 32), eil)
    S2f = S2.reshape(NC, X32, 128)
    out = _l2post(S2f[0], S2f[1], g2.reshape(X32, 128), rep32,
                  Wb2, bb2, Wb3, bb3)
    return out.reshape(N_PAD, 4)[:N]


# speedup vs baseline: 64.4109x; 1.3558x over previous
import functools

import jax
import jax.numpy as jnp
from jax import lax
from jax.experimental import pallas as pl
from jax.experimental.pallas import tpu as pltpu
from jax.experimental.pallas import tpu_sc as plsc

N = 100000
E = 1600000
IN_C, HID_C, OUT_C = 9, 32, 4

NC, NS = 2, 16
CH = 128
N_PAD = 100352
NB = 8

ECH = E // CH
WCH, WXT = ECH // (NC * NS), ECH % (NC * NS)
CPS = ECH // NC
TCH, TXT = CPS // NS, CPS % NS

X16 = N_PAD * 16 // 128
X32 = N_PAD * 32 // 128

_SC_PARAMS = pltpu.CompilerParams(use_tc_tiling_on_sc=False)


def _src_off(chunk):
    return (2 * chunk) * CH


def _dst_off(chunk):
    return (2 * chunk + 1) * CH


def _deg_body(eil_hbm, degp_hbm, acc, dstb, onesb, zb, isem, asem):
    c = lax.axis_index("c")
    s = lax.axis_index("s")
    w = c * NS + s

    def _zb(i, _):
        zb[pl.ds(i * 16, 16)] = jnp.zeros((16,), jnp.float32)
        return 0
    lax.fori_loop(0, zb.shape[0] // 16, _zb, 0)
    sl = N_PAD // NS
    pltpu.sync_copy(zb, acc.at[pl.ds(s * sl, sl)])
    for j in range(CH // 16):
        onesb[pl.ds(j * 16, 16)] = jnp.ones((16,), jnp.float32)
    plsc.subcore_barrier()

    base = w * WCH

    def _burst(ch0, m):
        din = [pltpu.async_copy(eil_hbm.at[pl.ds(_dst_off(ch0 + b), CH)],
                                dstb.at[b], isem.at[b]) for b in range(m)]
        dadd = []
        for b in range(m):
            din[b].wait()
            dadd.append(pltpu.async_copy(onesb, acc.at[dstb.at[b]],
                                         asem.at[b], add=True))
        for b in range(m):
            dadd[b].wait()

    r = WCH % NB
    if r:
        _burst(base, r)

    def _full(o, _):
        _burst(base + r + o * NB, NB)
        return 0
    lax.fori_loop(0, WCH // NB, _full, 0)

    @pl.when(w < WXT)
    def _extra():
        _burst((NC * NS) * WCH + w, 1)

    plsc.subcore_barrier()
    pltpu.sync_copy(acc.at[pl.ds(s * sl, sl)], degp_hbm.at[c, pl.ds(s * sl, sl)])


def _deg_partials(eil):
    mesh = plsc.VectorSubcoreMesh(core_axis_name="c", subcore_axis_name="s")
    return pl.kernel(
        _deg_body,
        out_type=jax.ShapeDtypeStruct((NC, N_PAD), jnp.float32),
        mesh=mesh,
        scratch_types=[
            pltpu.VMEM_SHARED((N_PAD,), jnp.float32),
            pltpu.VMEM((NB, CH), jnp.int32),
            pltpu.VMEM((CH,), jnp.float32),
            pltpu.VMEM((N_PAD // NS,), jnp.float32),
            pltpu.SemaphoreType.DMA((NB,)),
            pltpu.SemaphoreType.DMA((NB,)),
        ],
        compiler_params=_SC_PARAMS,
    )(eil)


def _scat_body(g_hbm, eil_hbm, out_hbm, acc, srcb, dstb, rows, zb,
               isem, gsem, ssem, D, dt):
    c = lax.axis_index("c")
    s = lax.axis_index("s")

    zr = zb.shape[0]
    zvec = 32 if dt == jnp.bfloat16 else 16

    def _zb(i, _):
        for k in range(D // zvec):
            zb[i, pl.ds(k * zvec, zvec)] = jnp.zeros((zvec,), dt)
        return 0
    lax.fori_loop(0, zr, _zb, 0)
    sl = N_PAD // NS

    def _zc(i, _):
        pltpu.sync_copy(zb, acc.at[pl.ds(s * sl + i * zr, zr)])
        return 0
    lax.fori_loop(0, sl // zr, _zc, 0)
    plsc.subcore_barrier()

    base = c * CPS + s * TCH

    def _burst(ch0, m):
        din = []
        for b in range(m):
            din.append((
                pltpu.async_copy(eil_hbm.at[pl.ds(_src_off(ch0 + b), CH)],
                                 srcb.at[b], isem.at[b]),
                pltpu.async_copy(eil_hbm.at[pl.ds(_dst_off(ch0 + b), CH)],
                                 dstb.at[b], isem.at[b]),
            ))
        dg = []
        for b in range(m):
            din[b][0].wait()
            dg.append(pltpu.async_copy(g_hbm.at[srcb.at[b]], rows.at[b],
                                       gsem.at[b]))
        ds_ = []
        for b in range(m):
            din[b][1].wait()
            dg[b].wait()
            ds_.append(pltpu.async_copy(rows.at[b], acc.at[dstb.at[b]],
                                        ssem.at[b], add=True))
        for b in range(m):
            ds_[b].wait()

    r = TCH % NB
    if r:
        _burst(base, r)

    def _full(o, _):
        _burst(base + r + o * NB, NB)
        return 0
    lax.fori_loop(0, TCH // NB, _full, 0)

    @pl.when(s < TXT)
    def _extra():
        _burst(c * CPS + NS * TCH + s, 1)

    plsc.subcore_barrier()
    pltpu.sync_copy(acc.at[pl.ds(s * sl, sl)],
                    out_hbm.at[c, pl.ds(s * sl, sl)])


def _scatter_rows(g, eil):
    D = g.shape[1]
    dt = g.dtype
    mesh = plsc.VectorSubcoreMesh(core_axis_name="c", subcore_axis_name="s")
    zr = 98
    return pl.kernel(
        functools.partial(_scat_body, D=D, dt=dt),
        out_type=jax.ShapeDtypeStruct((NC, N_PAD, D), dt),
        mesh=mesh,
        scratch_types=[
            pltpu.VMEM_SHARED((N_PAD, D), dt),
            pltpu.VMEM((NB, CH), jnp.int32),
            pltpu.VMEM((NB, CH), jnp.int32),
            pltpu.VMEM((NB, CH, D), dt),
            pltpu.VMEM((zr, D), dt),
            pltpu.SemaphoreType.DMA((NB,)),
            pltpu.SemaphoreType.DMA((NB,)),
            pltpu.SemaphoreType.DMA((NB,)),
        ],
        compiler_params=_SC_PARAMS,
    )(g, eil)


_GRID = 16
_NBLK = N_PAD // _GRID
_B16 = X16 // _GRID
_B32 = X32 // _GRID


def _dis_body(degp_ref, dis_ref):
    deg = degp_ref[0:1, :] + degp_ref[1:2, :] + 1.0
    dis_ref[...] = 1.0 / jnp.sqrt(deg)


def _dis(degp):
    return pl.pallas_call(
        _dis_body,
        grid=(_GRID,),
        in_specs=[pl.BlockSpec((2, _NBLK), lambda i: (0, i))],
        out_specs=pl.BlockSpec((1, _NBLK), lambda i: (0, i)),
        out_shape=jax.ShapeDtypeStruct((1, N_PAD), jnp.float32),
    )(degp)


def _gx_body(x_ref, rep_ref, gx_ref):
    gx_ref[...] = x_ref[...] * rep_ref[...]


def _gx(x_flat, rep16):
    return pl.pallas_call(
        _gx_body,
        grid=(_GRID,),
        in_specs=[
            pl.BlockSpec((_B16, 128), lambda i: (i, 0)),
            pl.BlockSpec((_B16, 128), lambda i: (i, 0)),
        ],
        out_specs=pl.BlockSpec((_B16, 128), lambda i: (i, 0)),
        out_shape=jax.ShapeDtypeStruct((X16, 128), jnp.float32),
    )(x_flat, rep16)


def _l1post_body(S1a_ref, S1b_ref, gx_ref, r16_ref, r32_ref, W_ref, b_ref,
                 g2_ref):
    t = r16_ref[...] * (S1a_ref[0] + S1b_ref[0] + gx_ref[...])
    h = jnp.maximum(
        jnp.dot(t, W_ref[...], preferred_element_type=jnp.float32)
        + b_ref[...], 0.0)
    g2 = (r32_ref[...] * h).astype(jnp.bfloat16)
    g2_ref[...] = g2.reshape(2 * _B16, 128)


def _l1post(S1r, gxf, rep16, rep32w, Wb1, bb1):
    return pl.pallas_call(
        _l1post_body,
        grid=(_GRID,),
        in_specs=[
            pl.BlockSpec((1, _B16, 128), lambda i: (0, i, 0)),
            pl.BlockSpec((1, _B16, 128), lambda i: (1, i, 0)),
            pl.BlockSpec((_B16, 128), lambda i: (i, 0)),
            pl.BlockSpec((_B16, 128), lambda i: (i, 0)),
            pl.BlockSpec((_B16, 256), lambda i: (i, 0)),
            pl.BlockSpec((128, 256), lambda i: (0, 0)),
            pl.BlockSpec((1, 256), lambda i: (0, 0)),
        ],
        out_specs=pl.BlockSpec((2 * _B16, 128), lambda i: (i, 0)),
        out_shape=jax.ShapeDtypeStruct((X32, 128), jnp.bfloat16),
    )(S1r, S1r, gxf, rep16, rep32w, Wb1, bb1)


def _l2post_body(S2a_ref, S2b_ref, g2_ref, r32_ref, W2_ref, b2_ref, Wl_ref,
                 bl_ref, out_ref):
    agg = (S2a_ref[0].astype(jnp.float32) + S2b_ref[0].astype(jnp.float32)
           + g2_ref[...].astype(jnp.float32))
    u = r32_ref[...] * agg
    h = jnp.maximum(
        jnp.dot(u, W2_ref[...], preferred_element_type=jnp.float32)
        + b2_ref[...], 0.0)
    out_ref[...] = (
        jnp.dot(h, Wl_ref[...], preferred_element_type=jnp.float32)
        + bl_ref[...])


def _l2post(S2r, g2f, rep32, Wb2, bb2, Wb3, bb3):
    return pl.pallas_call(
        _l2post_body,
        grid=(_GRID,),
        in_specs=[
            pl.BlockSpec((1, _B32, 128), lambda i: (0, i, 0)),
            pl.BlockSpec((1, _B32, 128), lambda i: (1, i, 0)),
            pl.BlockSpec((_B32, 128), lambda i: (i, 0)),
            pl.BlockSpec((_B32, 128), lambda i: (i, 0)),
            pl.BlockSpec((128, 128), lambda i: (0, 0)),
            pl.BlockSpec((1, 128), lambda i: (0, 0)),
            pl.BlockSpec((128, 16), lambda i: (0, 0)),
            pl.BlockSpec((1, 16), lambda i: (0, 0)),
        ],
        out_specs=pl.BlockSpec((_B32, 16), lambda i: (i, 0)),
        out_shape=jax.ShapeDtypeStruct((X32, 16), jnp.float32),
    )(S2r, S2r, g2f, rep32, Wb2, bb2, Wb3, bb3)


def _block_diag(W, k):
    m, n = W.shape
    out = jnp.zeros((k * m, k * n), W.dtype)
    for i in range(k):
        out = out.at[i * m:(i + 1) * m, i * n:(i + 1) * n].set(W)
    return out


def kernel(x, edge_index, W1, b1, W2, b2, Wl, bl):
    eil = jnp.swapaxes(edge_index.reshape(2, ECH, CH), 0, 1).reshape(2 * E)

    x_flat = jnp.pad(x, ((0, N_PAD - N), (0, 16 - IN_C))).reshape(X16, 128)
    W1p = jnp.pad(W1, ((0, 16 - IN_C), (0, 0)))
    Wb1 = _block_diag(W1p, 8)
    bb1 = jnp.tile(b1, 8).reshape(1, 256)
    Wb2 = _block_diag(W2, 4)
    bb2 = jnp.tile(b2, 4).reshape(1, 128)
    Wb3 = _block_diag(Wl, 4)
    bb3 = jnp.tile(bl, 4).reshape(1, 16)

    degp = _deg_partials(eil)
    dis = _dis(degp).reshape(N_PAD)
    rep16 = jnp.broadcast_to(dis[:, None], (N_PAD, 16)).reshape(X16, 128)
    rep32 = jnp.broadcast_to(dis[:, None], (N_PAD, 32)).reshape(X32, 128)

    gxf = _gx(x_flat, rep16)
    S1 = _scatter_rows(gxf.reshape(N_PAD, 16), eil)
    g2 = _l1post(S1.reshape(NC, X16, 128), gxf, rep16,
                 rep32.reshape(X16, 256), Wb1, bb1)

    S2 = _scatter_rows(g2.reshape(N_PAD, 32), eil)
    out = _l2post(S2.reshape(NC, X32, 128), g2, rep32,
                  Wb2, bb2, Wb3, bb3)
    return out.reshape(N_PAD, 4)[:N]
